# baseline (speedup 1.0000x reference)
import functools

import jax
import jax.numpy as jnp
from jax import lax
from jax.experimental import pallas as pl
from jax.experimental.pallas import tpu as pltpu
from jax.experimental.pallas import tpu_sc as plsc

VOCAB = 100000
DIM = 64
BATCH = 4096
SEQ = 200
NC = 2
NS = 16
NW = NC * NS
BW = BATCH // NW
L = 16
NPAIR = SEQ // 2


@functools.partial(
    pl.kernel,
    mesh=plsc.VectorSubcoreMesh(core_axis_name="c", subcore_axis_name="s"),
    out_type=jax.ShapeDtypeStruct((SEQ, 8, 32, 8, 128), jnp.float32),
    scratch_types=[
        pltpu.VMEM((BW, SEQ), jnp.int32),
        pltpu.VMEM((SEQ, BW), jnp.int32),
        pltpu.VMEM((BW, DIM), jnp.float32),
        pltpu.VMEM((BW, DIM), jnp.float32),
        pltpu.VMEM((8, 8, 133), jnp.float32),
        pltpu.VMEM((8, 8, 133), jnp.float32),
        pltpu.SemaphoreType.DMA,
        pltpu.SemaphoreType.DMA,
        pltpu.SemaphoreType.DMA,
        pltpu.SemaphoreType.DMA,
    ],
    compiler_params=pltpu.CompilerParams(
        use_tc_tiling_on_sc=False, needs_layout_passes=False),
)
def _gather_kernel(x_hbm, table_hbm, out_hbm, xbuf, ibuf, rows0, rows1,
                   tbuf0, tbuf1, sg0, sg1, ss0, ss1):
    wid = lax.axis_index("s") * NC + lax.axis_index("c")
    lanes = lax.iota(jnp.int32, L)

    pltpu.sync_copy(x_hbm.at[pl.ds(wid * BW, BW)], xbuf)

    def build_idx(s, _):
        for g in range(BW // L):
            vals = plsc.load_gather(
                xbuf, [g * L + lanes, jnp.full((L,), s, jnp.int32)])
            ibuf[s, pl.ds(g * L, L)] = vals
        return ()

    lax.fori_loop(0, SEQ, build_idx, ())

    def start_gather(s, rows, sem):
        pltpu.async_copy(table_hbm.at[ibuf.at[s]], rows, sem)

    dh_vecs = [(jnp.int32(dg * L) + lanes) // 8 for dg in range(DIM // L)]
    dl_vecs = [(jnp.int32(dg * L) + lanes) % 8 for dg in range(DIM // L)]

    def transpose(rows, tbuf):
        def tbody(i, _):
            pairs = [(8 * i + bi, dg)
                     for bi in range(8) for dg in range(DIM // L)]
            vals = [rows[b, pl.ds(dg * L, L)] for b, dg in pairs]
            for (b, dg), v in zip(pairs, vals):
                plsc.store_scatter(
                    tbuf, [dh_vecs[dg], dl_vecs[dg],
                           jnp.full((L,), b, jnp.int32)], v)
            return ()

        lax.fori_loop(0, BW // 8, tbody, ())

    def start_store(s, tbuf, sem):
        pltpu.async_copy(
            tbuf.at[:, :, pl.ds(0, 128)], out_hbm.at[s, :, wid, :, :], sem)

    def wait_store(s, tbuf, sem):
        pltpu.make_async_copy(
            tbuf.at[:, :, pl.ds(0, 128)], out_hbm.at[s, :, wid, :, :],
            sem).wait()

    def wait_gather(s, rows, sem):
        pltpu.make_async_copy(table_hbm.at[ibuf.at[s]], rows, sem).wait()

    start_gather(0, rows0, sg0)

    def body(j, _):
        s0 = 2 * j
        s1 = s0 + 1
        start_gather(s1, rows1, sg1)

        wait_gather(s0, rows0, sg0)

        @pl.when(j > 0)
        def _():
            wait_store(s0 - 2, tbuf0, ss0)

        transpose(rows0, tbuf0)
        start_store(s0, tbuf0, ss0)

        @pl.when(j < NPAIR - 1)
        def _():
            start_gather(s0 + 2, rows0, sg0)

        wait_gather(s1, rows1, sg1)

        @pl.when(j > 0)
        def _():
            wait_store(s1 - 2, tbuf1, ss1)

        transpose(rows1, tbuf1)
        start_store(s1, tbuf1, ss1)
        return ()

    lax.fori_loop(0, NPAIR, body, ())

    wait_store(SEQ - 2, tbuf0, ss0)
    wait_store(SEQ - 1, tbuf1, ss1)


def kernel(x, table):
    out5 = _gather_kernel(x, table)
    return jnp.transpose(out5, (2, 4, 0, 1, 3)).reshape(BATCH, SEQ, DIM)


# speedup vs baseline: 1.0131x; 1.0131x over previous
import functools

import jax
import jax.numpy as jnp
from jax import lax
from jax.experimental import pallas as pl
from jax.experimental.pallas import tpu as pltpu
from jax.experimental.pallas import tpu_sc as plsc

VOCAB = 100000
DIM = 64
BATCH = 4096
SEQ = 200
NC = 2
NS = 16
NW = NC * NS
BW = BATCH // NW
L = 16
NPAIR = SEQ // 2


@functools.partial(
    pl.kernel,
    mesh=plsc.VectorSubcoreMesh(core_axis_name="c", subcore_axis_name="s"),
    out_type=jax.ShapeDtypeStruct((SEQ, 8, 32, 8, 128), jnp.float32),
    scratch_types=[
        pltpu.VMEM((BW, SEQ), jnp.int32),
        pltpu.VMEM((SEQ, BW), jnp.int32),
        pltpu.VMEM((BW, DIM), jnp.float32),
        pltpu.VMEM((BW, DIM), jnp.float32),
        pltpu.VMEM((8, 8, 133), jnp.float32),
        pltpu.VMEM((8, 8, 133), jnp.float32),
        pltpu.SemaphoreType.DMA,
        pltpu.SemaphoreType.DMA,
        pltpu.SemaphoreType.DMA,
        pltpu.SemaphoreType.DMA,
    ],
    compiler_params=pltpu.CompilerParams(
        use_tc_tiling_on_sc=False, needs_layout_passes=False),
)
def _gather_kernel(x_hbm, table_hbm, out_hbm, xbuf, ibuf, rows0, rows1,
                   tbuf0, tbuf1, sg0, sg1, ss0, ss1):
    wid = lax.axis_index("s") * NC + lax.axis_index("c")
    lanes = lax.iota(jnp.int32, L)

    pltpu.sync_copy(x_hbm.at[pl.ds(wid * BW, BW)], xbuf)

    def build_idx(s, _):
        for g in range(BW // L):
            vals = plsc.load_gather(
                xbuf, [g * L + lanes, jnp.full((L,), s, jnp.int32)])
            ibuf[s, pl.ds(g * L, L)] = vals
        return ()

    lax.fori_loop(0, SEQ, build_idx, ())

    def start_gather(s, rows, sem):
        pltpu.async_copy(table_hbm.at[ibuf.at[s]], rows, sem)

    dh_vecs = [(jnp.int32(dg * L) + lanes) // 8 for dg in range(DIM // L)]
    dl_vecs = [(jnp.int32(dg * L) + lanes) % 8 for dg in range(DIM // L)]

    def transpose(rows, tbuf):
        def tbody(i, _):
            pairs = [(4 * i + bi, dg)
                     for bi in range(4) for dg in range(DIM // L)]
            vals = [rows[b, pl.ds(dg * L, L)] for b, dg in pairs]
            for (b, dg), v in zip(pairs, vals):
                plsc.store_scatter(
                    tbuf, [dh_vecs[dg], dl_vecs[dg],
                           jnp.full((L,), b, jnp.int32)], v)
            return ()

        lax.fori_loop(0, BW // 4, tbody, ())

    def start_store(s, tbuf, sem):
        pltpu.async_copy(
            tbuf.at[:, :, pl.ds(0, 128)], out_hbm.at[s, :, wid, :, :], sem)

    def wait_store(s, tbuf, sem):
        pltpu.make_async_copy(
            tbuf.at[:, :, pl.ds(0, 128)], out_hbm.at[s, :, wid, :, :],
            sem).wait()

    def wait_gather(s, rows, sem):
        pltpu.make_async_copy(table_hbm.at[ibuf.at[s]], rows, sem).wait()

    start_gather(0, rows0, sg0)

    def body(j, _):
        s0 = 2 * j
        s1 = s0 + 1
        start_gather(s1, rows1, sg1)

        wait_gather(s0, rows0, sg0)

        @pl.when(j > 0)
        def _():
            wait_store(s0 - 2, tbuf0, ss0)

        transpose(rows0, tbuf0)
        start_store(s0, tbuf0, ss0)

        @pl.when(j < NPAIR - 1)
        def _():
            start_gather(s0 + 2, rows0, sg0)

        wait_gather(s1, rows1, sg1)

        @pl.when(j > 0)
        def _():
            wait_store(s1 - 2, tbuf1, ss1)

        transpose(rows1, tbuf1)
        start_store(s1, tbuf1, ss1)
        return ()

    lax.fori_loop(0, NPAIR, body, ())

    wait_store(SEQ - 2, tbuf0, ss0)
    wait_store(SEQ - 1, tbuf1, ss1)


def kernel(x, table):
    out5 = _gather_kernel(x, table)
    return jnp.transpose(out5, (2, 4, 0, 1, 3)).reshape(BATCH, SEQ, DIM)


# speedup vs baseline: 1.0202x; 1.0069x over previous
import functools

import jax
import jax.numpy as jnp
from jax import lax
from jax.experimental import pallas as pl
from jax.experimental.pallas import tpu as pltpu
from jax.experimental.pallas import tpu_sc as plsc

VOCAB = 100000
DIM = 64
BATCH = 4096
SEQ = 200
NC = 2
NS = 16
NW = NC * NS
BW = BATCH // NW
L = 16
NPAIR = SEQ // 2


@functools.partial(
    pl.kernel,
    mesh=plsc.VectorSubcoreMesh(core_axis_name="c", subcore_axis_name="s"),
    out_type=jax.ShapeDtypeStruct((SEQ, 8, 32, 8, 128), jnp.float32),
    scratch_types=[
        pltpu.VMEM((BW, SEQ), jnp.int32),
        pltpu.VMEM((SEQ * BW,), jnp.int32),
        pltpu.VMEM((2 * BW, DIM), jnp.float32),
        pltpu.VMEM((2 * BW, DIM), jnp.float32),
        pltpu.VMEM((8, 8, 133), jnp.float32),
        pltpu.VMEM((8, 8, 133), jnp.float32),
        pltpu.VMEM((8, 8, 133), jnp.float32),
        pltpu.VMEM((8, 8, 133), jnp.float32),
        pltpu.SemaphoreType.DMA,
        pltpu.SemaphoreType.DMA,
        pltpu.SemaphoreType.DMA,
        pltpu.SemaphoreType.DMA,
        pltpu.SemaphoreType.DMA,
        pltpu.SemaphoreType.DMA,
    ],
    compiler_params=pltpu.CompilerParams(
        use_tc_tiling_on_sc=False, needs_layout_passes=False),
)
def _gather_kernel(x_hbm, table_hbm, out_hbm, xbuf, ibuf, rows0, rows1,
                   tbuf0, tbuf1, tbuf2, tbuf3, sg0, sg1,
                   ss0, ss1, ss2, ss3):
    wid = lax.axis_index("s") * NC + lax.axis_index("c")
    lanes = lax.iota(jnp.int32, L)

    pltpu.sync_copy(x_hbm.at[pl.ds(wid * BW, BW)], xbuf)

    def build_idx(s, _):
        for g in range(BW // L):
            vals = plsc.load_gather(
                xbuf, [g * L + lanes, jnp.full((L,), s, jnp.int32)])
            ibuf[pl.ds(s * BW + g * L, L)] = vals
        return ()

    lax.fori_loop(0, SEQ, build_idx, ())

    def start_gather(s, rows, sem):
        pltpu.async_copy(
            table_hbm.at[ibuf.at[pl.ds(s * BW, 2 * BW)]], rows, sem)

    def wait_gather(s, rows, sem):
        pltpu.make_async_copy(
            table_hbm.at[ibuf.at[pl.ds(s * BW, 2 * BW)]], rows, sem).wait()

    dh_vecs = [(jnp.int32(dg * L) + lanes) // 8 for dg in range(DIM // L)]
    dl_vecs = [(jnp.int32(dg * L) + lanes) % 8 for dg in range(DIM // L)]

    def transpose(rows, half, tbuf):
        def tbody(i, _):
            pairs = [(4 * i + bi, dg)
                     for bi in range(4) for dg in range(DIM // L)]
            vals = [rows[half * BW + b, pl.ds(dg * L, L)]
                    for b, dg in pairs]
            for (b, dg), v in zip(pairs, vals):
                plsc.store_scatter(
                    tbuf, [dh_vecs[dg], dl_vecs[dg],
                           jnp.full((L,), b, jnp.int32)], v)
            return ()

        lax.fori_loop(0, BW // 4, tbody, ())

    def start_store(s, tbuf, sem):
        pltpu.async_copy(
            tbuf.at[:, :, pl.ds(0, 128)], out_hbm.at[s, :, wid, :, :], sem)

    def wait_store(s, tbuf, sem):
        pltpu.make_async_copy(
            tbuf.at[:, :, pl.ds(0, 128)], out_hbm.at[s, :, wid, :, :],
            sem).wait()

    tbufs = [tbuf0, tbuf1, tbuf2, tbuf3]
    sss = [ss0, ss1, ss2, ss3]

    start_gather(0, rows0, sg0)

    NQUAD = SEQ // 4

    def body(j, _):
        s0 = 4 * j
        start_gather(s0 + 2, rows1, sg1)

        wait_gather(s0, rows0, sg0)
        for k in (0, 1):
            @pl.when(j > 0)
            def _(k=k):
                wait_store(s0 + k - 4, tbufs[k], sss[k])
            transpose(rows0, k, tbufs[k])
            start_store(s0 + k, tbufs[k], sss[k])

        @pl.when(j < NQUAD - 1)
        def _():
            start_gather(s0 + 4, rows0, sg0)

        wait_gather(s0 + 2, rows1, sg1)
        for k in (2, 3):
            @pl.when(j > 0)
            def _(k=k):
                wait_store(s0 + k - 4, tbufs[k], sss[k])
            transpose(rows1, k - 2, tbufs[k])
            start_store(s0 + k, tbufs[k], sss[k])
        return ()

    lax.fori_loop(0, NQUAD, body, ())

    for k in range(4):
        wait_store(SEQ - 4 + k, tbufs[k], sss[k])


def kernel(x, table):
    out5 = _gather_kernel(x, table)
    return jnp.transpose(out5, (2, 4, 0, 1, 3)).reshape(BATCH, SEQ, DIM)
